# baseline (speedup 1.0000x reference)
import functools

import jax
import jax.numpy as jnp
from jax import lax
from jax.experimental import pallas as pl
from jax.experimental.pallas import tpu as pltpu
from jax.experimental.pallas import tpu_sc as plsc

_NC = 2
_NS = 16
_NW = _NC * _NS
_L = 16

_CSEG = 2000
_GRP = _CSEG // _L


def _vsqrt(x):
    x = jnp.maximum(x, jnp.float32(1e-35))
    i = plsc.bitcast(x, jnp.int32)
    i = jnp.int32(0x5F3759DF) - (i >> 1)
    y = plsc.bitcast(i, jnp.float32)
    hx = jnp.float32(0.5) * x
    y = y * (jnp.float32(1.5) - hx * y * y)
    y = y * (jnp.float32(1.5) - hx * y * y)
    return x * y


def _make_sc_kernel(S):
    assert S % _CSEG == 0
    nchunk = S // _CSEG
    mesh = plsc.VectorSubcoreMesh(core_axis_name="c", subcore_axis_name="s")

    @functools.partial(
        pl.kernel,
        mesh=mesh,
        out_type=jax.ShapeDtypeStruct((_NW, _L), jnp.float32),
        compiler_params=pltpu.CompilerParams(
            needs_layout_passes=False, use_tc_tiling_on_sc=False
        ),
        scratch_types=[
            pltpu.VMEM((_CSEG * 3, 3), jnp.float32),
            pltpu.VMEM((_CSEG * 3,), jnp.float32),
            pltpu.VMEM((_CSEG * 3, 1), jnp.float32),
            pltpu.VMEM((_L,), jnp.float32),
        ],
    )
    def sc_kernel(inp_hbm, tgt_hbm, msk_hbm, out_hbm, tgt_v, inp_v, msk_v, acc_v):
        cid = lax.axis_index("c")
        sid = lax.axis_index("s")
        wid = sid * _NC + cid
        cnt = (nchunk - wid + _NW - 1) // _NW

        iota = lax.iota(jnp.int32, _L)
        three = iota * 3
        c0 = jnp.zeros((_L,), jnp.int32)
        c1 = c0 + 1
        c2 = c0 + 2
        third = jnp.float32(1.0 / 3.0)
        eps = jnp.float32(1e-6)

        def chunk_body(k, acc):
            s0 = (wid + k * _NW) * _CSEG
            pltpu.sync_copy(tgt_hbm.at[pl.ds(s0 * 3, _CSEG * 3), :], tgt_v)
            pltpu.sync_copy(inp_hbm.at[pl.ds(s0 * 3, _CSEG * 3)], inp_v)
            pltpu.sync_copy(msk_hbm.at[pl.ds(s0 * 3, _CSEG * 3), :], msk_v)

            def grp_body(j, a):
                r0 = three + j * (3 * _L)
                r1 = r0 + 1
                r2 = r0 + 2
                t0x = plsc.load_gather(tgt_v, [r0, c0])
                t0y = plsc.load_gather(tgt_v, [r0, c1])
                t0z = plsc.load_gather(tgt_v, [r0, c2])
                t1x = plsc.load_gather(tgt_v, [r1, c0])
                t1y = plsc.load_gather(tgt_v, [r1, c1])
                t1z = plsc.load_gather(tgt_v, [r1, c2])
                t2x = plsc.load_gather(tgt_v, [r2, c0])
                t2y = plsc.load_gather(tgt_v, [r2, c1])
                t2z = plsc.load_gather(tgt_v, [r2, c2])
                m0 = plsc.load_gather(msk_v, [r0, c0])
                m1 = plsc.load_gather(msk_v, [r1, c0])
                m2 = plsc.load_gather(msk_v, [r2, c0])
                x0 = plsc.load_gather(inp_v, [r0])
                x1 = plsc.load_gather(inp_v, [r1])
                x2 = plsc.load_gather(inp_v, [r2])

                ax = t0x - t1x
                ay = t0y - t1y
                az = t0z - t1z
                bx = t0x - t2x
                by = t0y - t2y
                bz = t0z - t2z
                cx = t1x - t2x
                cy = t1y - t2y
                cz = t1z - t2z
                d01 = _vsqrt(ax * ax + ay * ay + az * az)
                d02 = _vsqrt(bx * bx + by * by + bz * bz)
                d12 = _vsqrt(cx * cx + cy * cy + cz * cz)

                e0 = (m0 * m1) * (x0 - d01)
                e1 = (m0 * m2) * (x1 - d02)
                e2 = (m1 * m2) * (x2 - d12)
                r = e0 * e0 + e1 * e1 + e2 * e2
                return a + _vsqrt(r * third + eps)

            return lax.fori_loop(0, _GRP, grp_body, acc)

        acc = lax.fori_loop(0, cnt, chunk_body, jnp.zeros((_L,), jnp.float32))
        acc_v[...] = acc
        pltpu.sync_copy(acc_v, out_hbm.at[wid])

    return sc_kernel


def kernel(inputs, target, mask, indices):
    S = indices.shape[0]
    partials = _make_sc_kernel(S)(inputs, target, mask)
    return jnp.sum(partials) / jnp.float32(S)


# speedup vs baseline: 38.3480x; 38.3480x over previous
import functools

import jax
import jax.numpy as jnp
from jax import lax
from jax.experimental import pallas as pl
from jax.experimental.pallas import tpu as pltpu
from jax.experimental.pallas import tpu_sc as plsc

_NC = 2
_NS = 16
_NW = _NC * _NS
_L = 16

_CSEG = 4000
_GRP = _CSEG // _L


def _vsqrt(x):
    x = jnp.maximum(x, jnp.float32(1e-35))
    i = plsc.bitcast(x, jnp.int32)
    i = jnp.int32(0x5F3759DF) - (i >> 1)
    y = plsc.bitcast(i, jnp.float32)
    hx = jnp.float32(0.5) * x
    y = y * (jnp.float32(1.5) - hx * y * y)
    y = y * (jnp.float32(1.5) - hx * y * y)
    return x * y


def _make_sc_kernel(S):
    assert S % _CSEG == 0
    nchunk = S // _CSEG
    mesh = plsc.VectorSubcoreMesh(core_axis_name="c", subcore_axis_name="s")

    @functools.partial(
        pl.kernel,
        mesh=mesh,
        out_type=jax.ShapeDtypeStruct((_NW, _L), jnp.float32),
        compiler_params=pltpu.CompilerParams(
            needs_layout_passes=False, use_tc_tiling_on_sc=False
        ),
        scratch_types=[
            pltpu.VMEM((_CSEG * 3,), jnp.float32),
            pltpu.VMEM((_CSEG * 3,), jnp.float32),
            pltpu.VMEM((_CSEG * 3,), jnp.float32),
            pltpu.VMEM((_CSEG * 3,), jnp.float32),
            pltpu.VMEM((_CSEG * 3,), jnp.float32),
            pltpu.VMEM((_L,), jnp.float32),
        ],
    )
    def sc_kernel(
        inp_hbm, tx_hbm, ty_hbm, tz_hbm, msk_hbm, out_hbm,
        tx_v, ty_v, tz_v, inp_v, msk_v, acc_v,
    ):
        cid = lax.axis_index("c")
        sid = lax.axis_index("s")
        wid = sid * _NC + cid
        cnt = (nchunk - wid + _NW - 1) // _NW

        iota = lax.iota(jnp.int32, _L)
        three = iota * 3
        third = jnp.float32(1.0 / 3.0)
        eps = jnp.float32(1e-6)

        def chunk_body(k, acc):
            s0 = (wid + k * _NW) * _CSEG
            rows = pl.ds(s0 * 3, _CSEG * 3)
            pltpu.sync_copy(tx_hbm.at[rows], tx_v)
            pltpu.sync_copy(ty_hbm.at[rows], ty_v)
            pltpu.sync_copy(tz_hbm.at[rows], tz_v)
            pltpu.sync_copy(inp_hbm.at[rows], inp_v)
            pltpu.sync_copy(msk_hbm.at[rows], msk_v)

            def grp_body(j, a):
                r0 = three + j * (3 * _L)
                r1 = r0 + 1
                r2 = r0 + 2
                t0x = plsc.load_gather(tx_v, [r0])
                t0y = plsc.load_gather(ty_v, [r0])
                t0z = plsc.load_gather(tz_v, [r0])
                t1x = plsc.load_gather(tx_v, [r1])
                t1y = plsc.load_gather(ty_v, [r1])
                t1z = plsc.load_gather(tz_v, [r1])
                t2x = plsc.load_gather(tx_v, [r2])
                t2y = plsc.load_gather(ty_v, [r2])
                t2z = plsc.load_gather(tz_v, [r2])
                m0 = plsc.load_gather(msk_v, [r0])
                m1 = plsc.load_gather(msk_v, [r1])
                m2 = plsc.load_gather(msk_v, [r2])
                x0 = plsc.load_gather(inp_v, [r0])
                x1 = plsc.load_gather(inp_v, [r1])
                x2 = plsc.load_gather(inp_v, [r2])

                ax = t0x - t1x
                ay = t0y - t1y
                az = t0z - t1z
                bx = t0x - t2x
                by = t0y - t2y
                bz = t0z - t2z
                cx = t1x - t2x
                cy = t1y - t2y
                cz = t1z - t2z
                d01 = _vsqrt(ax * ax + ay * ay + az * az)
                d02 = _vsqrt(bx * bx + by * by + bz * bz)
                d12 = _vsqrt(cx * cx + cy * cy + cz * cz)

                e0 = (m0 * m1) * (x0 - d01)
                e1 = (m0 * m2) * (x1 - d02)
                e2 = (m1 * m2) * (x2 - d12)
                r = e0 * e0 + e1 * e1 + e2 * e2
                return a + _vsqrt(r * third + eps)

            return lax.fori_loop(0, _GRP, grp_body, acc)

        acc = lax.fori_loop(0, cnt, chunk_body, jnp.zeros((_L,), jnp.float32))
        acc_v[...] = acc
        pltpu.sync_copy(acc_v, out_hbm.at[wid])

    return sc_kernel


def kernel(inputs, target, mask, indices):
    S = indices.shape[0]
    tx = target[:, 0]
    ty = target[:, 1]
    tz = target[:, 2]
    partials = _make_sc_kernel(S)(inputs, tx, ty, tz, mask.reshape(-1))
    return jnp.sum(partials) / jnp.float32(S)


# speedup vs baseline: 40.4846x; 1.0557x over previous
import functools

import jax
import jax.numpy as jnp
from jax import lax
from jax.experimental import pallas as pl
from jax.experimental.pallas import tpu as pltpu
from jax.experimental.pallas import tpu_sc as plsc

_NC = 2
_NS = 16
_NW = _NC * _NS
_L = 16

_CSEG = 4000
_GRP = _CSEG // _L


def _vsqrt(x):
    x = jnp.maximum(x, jnp.float32(1e-35))
    i = plsc.bitcast(x, jnp.int32)
    i = jnp.int32(0x5F3759DF) - (i >> 1)
    y = plsc.bitcast(i, jnp.float32)
    hx = jnp.float32(0.5) * x
    y = y * (jnp.float32(1.5) - hx * y * y)
    y = y * (jnp.float32(1.5) - hx * y * y)
    return x * y


def _make_sc_kernel(S):
    assert S % _CSEG == 0
    nchunk = S // _CSEG
    mesh = plsc.VectorSubcoreMesh(core_axis_name="c", subcore_axis_name="s")

    @functools.partial(
        pl.kernel,
        mesh=mesh,
        out_type=jax.ShapeDtypeStruct((_NW, _L), jnp.float32),
        compiler_params=pltpu.CompilerParams(
            needs_layout_passes=False, use_tc_tiling_on_sc=False
        ),
        scratch_types=[
            pltpu.VMEM((_CSEG * 3,), jnp.float32),
            pltpu.VMEM((_CSEG * 3,), jnp.float32),
            pltpu.VMEM((_CSEG * 3,), jnp.float32),
            pltpu.VMEM((_CSEG * 3,), jnp.float32),
            pltpu.VMEM((_CSEG * 3,), jnp.float32),
            pltpu.VMEM((_L,), jnp.float32),
            pltpu.SemaphoreType.DMA,
        ],
    )
    def sc_kernel(
        inp_hbm, tx_hbm, ty_hbm, tz_hbm, msk_hbm, out_hbm,
        tx_v, ty_v, tz_v, inp_v, msk_v, acc_v, sem,
    ):
        cid = lax.axis_index("c")
        sid = lax.axis_index("s")
        wid = sid * _NC + cid
        cnt = (nchunk - wid + _NW - 1) // _NW

        iota = lax.iota(jnp.int32, _L)
        three = iota * 3
        third = jnp.float32(1.0 / 3.0)
        eps = jnp.float32(1e-6)

        def chunk_body(k, acc):
            s0 = (wid + k * _NW) * _CSEG
            rows = pl.ds(s0 * 3, _CSEG * 3)
            h0 = pltpu.async_copy(tx_hbm.at[rows], tx_v, sem)
            h1 = pltpu.async_copy(ty_hbm.at[rows], ty_v, sem)
            h2 = pltpu.async_copy(tz_hbm.at[rows], tz_v, sem)
            h3 = pltpu.async_copy(inp_hbm.at[rows], inp_v, sem)
            h4 = pltpu.async_copy(msk_hbm.at[rows], msk_v, sem)
            h0.wait()
            h1.wait()
            h2.wait()
            h3.wait()
            h4.wait()

            def grp_body(j, a):
                r0 = three + j * (3 * _L)
                r1 = r0 + 1
                r2 = r0 + 2
                t0x = plsc.load_gather(tx_v, [r0])
                t0y = plsc.load_gather(ty_v, [r0])
                t0z = plsc.load_gather(tz_v, [r0])
                t1x = plsc.load_gather(tx_v, [r1])
                t1y = plsc.load_gather(ty_v, [r1])
                t1z = plsc.load_gather(tz_v, [r1])
                t2x = plsc.load_gather(tx_v, [r2])
                t2y = plsc.load_gather(ty_v, [r2])
                t2z = plsc.load_gather(tz_v, [r2])
                m0 = plsc.load_gather(msk_v, [r0])
                m1 = plsc.load_gather(msk_v, [r1])
                m2 = plsc.load_gather(msk_v, [r2])
                x0 = plsc.load_gather(inp_v, [r0])
                x1 = plsc.load_gather(inp_v, [r1])
                x2 = plsc.load_gather(inp_v, [r2])

                ax = t0x - t1x
                ay = t0y - t1y
                az = t0z - t1z
                bx = t0x - t2x
                by = t0y - t2y
                bz = t0z - t2z
                cx = t1x - t2x
                cy = t1y - t2y
                cz = t1z - t2z
                d01 = _vsqrt(ax * ax + ay * ay + az * az)
                d02 = _vsqrt(bx * bx + by * by + bz * bz)
                d12 = _vsqrt(cx * cx + cy * cy + cz * cz)

                e0 = (m0 * m1) * (x0 - d01)
                e1 = (m0 * m2) * (x1 - d02)
                e2 = (m1 * m2) * (x2 - d12)
                r = e0 * e0 + e1 * e1 + e2 * e2
                return a + _vsqrt(r * third + eps)

            return lax.fori_loop(0, _GRP, grp_body, acc)

        acc = lax.fori_loop(0, cnt, chunk_body, jnp.zeros((_L,), jnp.float32))
        acc_v[...] = acc
        pltpu.sync_copy(acc_v, out_hbm.at[wid])

    return sc_kernel


def kernel(inputs, target, mask, indices):
    S = indices.shape[0]
    tx = target[:, 0]
    ty = target[:, 1]
    tz = target[:, 2]
    partials = _make_sc_kernel(S)(inputs, tx, ty, tz, mask.reshape(-1))
    return jnp.sum(partials) / jnp.float32(S)


# speedup vs baseline: 40.9283x; 1.0110x over previous
import functools

import jax
import jax.numpy as jnp
from jax import lax
from jax.experimental import pallas as pl
from jax.experimental.pallas import tpu as pltpu
from jax.experimental.pallas import tpu_sc as plsc

_NC = 2
_NS = 16
_NW = _NC * _NS
_L = 16

_CSEG = 8000
_GRP = _CSEG // _L


def _vsqrt(x):
    x = jnp.maximum(x, jnp.float32(1e-35))
    i = plsc.bitcast(x, jnp.int32)
    i = jnp.int32(0x5F3759DF) - (i >> 1)
    y = plsc.bitcast(i, jnp.float32)
    hx = jnp.float32(0.5) * x
    y = y * (jnp.float32(1.5) - hx * y * y)
    y = y * (jnp.float32(1.5) - hx * y * y)
    return x * y


def _make_sc_kernel(S):
    assert S % _CSEG == 0
    nchunk = S // _CSEG
    mesh = plsc.VectorSubcoreMesh(core_axis_name="c", subcore_axis_name="s")

    @functools.partial(
        pl.kernel,
        mesh=mesh,
        out_type=jax.ShapeDtypeStruct((_NW, _L), jnp.float32),
        compiler_params=pltpu.CompilerParams(
            needs_layout_passes=False, use_tc_tiling_on_sc=False
        ),
        scratch_types=[
            pltpu.VMEM((_CSEG * 3,), jnp.float32),
            pltpu.VMEM((_CSEG * 3,), jnp.float32),
            pltpu.VMEM((_CSEG * 3,), jnp.float32),
            pltpu.VMEM((_CSEG * 3,), jnp.float32),
            pltpu.VMEM((_CSEG * 3,), jnp.float32),
            pltpu.VMEM((_L,), jnp.float32),
            pltpu.SemaphoreType.DMA,
        ],
    )
    def sc_kernel(
        inp_hbm, tx_hbm, ty_hbm, tz_hbm, msk_hbm, out_hbm,
        tx_v, ty_v, tz_v, inp_v, msk_v, acc_v, sem,
    ):
        cid = lax.axis_index("c")
        sid = lax.axis_index("s")
        wid = sid * _NC + cid
        cnt = (nchunk - wid + _NW - 1) // _NW

        iota = lax.iota(jnp.int32, _L)
        three = iota * 3
        third = jnp.float32(1.0 / 3.0)
        eps = jnp.float32(1e-6)

        def chunk_body(k, acc):
            s0 = (wid + k * _NW) * _CSEG
            rows = pl.ds(s0 * 3, _CSEG * 3)
            h0 = pltpu.async_copy(tx_hbm.at[rows], tx_v, sem)
            h1 = pltpu.async_copy(ty_hbm.at[rows], ty_v, sem)
            h2 = pltpu.async_copy(tz_hbm.at[rows], tz_v, sem)
            h3 = pltpu.async_copy(inp_hbm.at[rows], inp_v, sem)
            h4 = pltpu.async_copy(msk_hbm.at[rows], msk_v, sem)
            h0.wait()
            h1.wait()
            h2.wait()
            h3.wait()
            h4.wait()

            def grp_body(j, a):
                r0 = three + j * (3 * _L)
                r1 = r0 + 1
                r2 = r0 + 2
                t0x = plsc.load_gather(tx_v, [r0])
                t0y = plsc.load_gather(ty_v, [r0])
                t0z = plsc.load_gather(tz_v, [r0])
                t1x = plsc.load_gather(tx_v, [r1])
                t1y = plsc.load_gather(ty_v, [r1])
                t1z = plsc.load_gather(tz_v, [r1])
                t2x = plsc.load_gather(tx_v, [r2])
                t2y = plsc.load_gather(ty_v, [r2])
                t2z = plsc.load_gather(tz_v, [r2])
                m0 = plsc.load_gather(msk_v, [r0])
                m1 = plsc.load_gather(msk_v, [r1])
                m2 = plsc.load_gather(msk_v, [r2])
                x0 = plsc.load_gather(inp_v, [r0])
                x1 = plsc.load_gather(inp_v, [r1])
                x2 = plsc.load_gather(inp_v, [r2])

                ax = t0x - t1x
                ay = t0y - t1y
                az = t0z - t1z
                bx = t0x - t2x
                by = t0y - t2y
                bz = t0z - t2z
                cx = t1x - t2x
                cy = t1y - t2y
                cz = t1z - t2z
                d01 = _vsqrt(ax * ax + ay * ay + az * az)
                d02 = _vsqrt(bx * bx + by * by + bz * bz)
                d12 = _vsqrt(cx * cx + cy * cy + cz * cz)

                e0 = (m0 * m1) * (x0 - d01)
                e1 = (m0 * m2) * (x1 - d02)
                e2 = (m1 * m2) * (x2 - d12)
                r = e0 * e0 + e1 * e1 + e2 * e2
                return a + _vsqrt(r * third + eps)

            return lax.fori_loop(0, _GRP, grp_body, acc)

        acc = lax.fori_loop(0, cnt, chunk_body, jnp.zeros((_L,), jnp.float32))
        acc_v[...] = acc
        pltpu.sync_copy(acc_v, out_hbm.at[wid])

    return sc_kernel


def kernel(inputs, target, mask, indices):
    S = indices.shape[0]
    tx = target[:, 0]
    ty = target[:, 1]
    tz = target[:, 2]
    partials = _make_sc_kernel(S)(inputs, tx, ty, tz, mask.reshape(-1))
    return jnp.sum(partials) / jnp.float32(S)


# speedup vs baseline: 42.5793x; 1.0403x over previous
import functools

import jax
import jax.numpy as jnp
from jax import lax
from jax.experimental import pallas as pl
from jax.experimental.pallas import tpu as pltpu
from jax.experimental.pallas import tpu_sc as plsc

_NC = 2
_NS = 16
_NW = _NC * _NS
_L = 16

_CSEG = 4000
_GRP = _CSEG // _L


def _vsqrt(x):
    x = jnp.maximum(x, jnp.float32(1e-35))
    i = plsc.bitcast(x, jnp.int32)
    i = jnp.int32(0x5F3759DF) - (i >> 1)
    y = plsc.bitcast(i, jnp.float32)
    hx = jnp.float32(0.5) * x
    y = y * (jnp.float32(1.5) - hx * y * y)
    y = y * (jnp.float32(1.5) - hx * y * y)
    return x * y


def _make_sc_kernel(S):
    assert S % _CSEG == 0
    nchunk = S // _CSEG
    assert nchunk >= _NW
    maxc = (nchunk + _NW - 1) // _NW
    mesh = plsc.VectorSubcoreMesh(core_axis_name="c", subcore_axis_name="s")

    @functools.partial(
        pl.kernel,
        mesh=mesh,
        out_type=jax.ShapeDtypeStruct((_NW, _L), jnp.float32),
        compiler_params=pltpu.CompilerParams(
            needs_layout_passes=False, use_tc_tiling_on_sc=False
        ),
        scratch_types=[
            pltpu.VMEM((_CSEG * 3,), jnp.float32),
            pltpu.VMEM((_CSEG * 3,), jnp.float32),
            pltpu.VMEM((_CSEG * 3,), jnp.float32),
            pltpu.VMEM((_CSEG * 3,), jnp.float32),
            pltpu.VMEM((_CSEG * 3,), jnp.float32),
            pltpu.VMEM((_CSEG * 3,), jnp.float32),
            pltpu.VMEM((_CSEG * 3,), jnp.float32),
            pltpu.VMEM((_CSEG * 3,), jnp.float32),
            pltpu.VMEM((_CSEG * 3,), jnp.float32),
            pltpu.VMEM((_CSEG * 3,), jnp.float32),
            pltpu.VMEM((_L,), jnp.float32),
            pltpu.SemaphoreType.DMA,
            pltpu.SemaphoreType.DMA,
        ],
    )
    def sc_kernel(
        inp_hbm, tx_hbm, ty_hbm, tz_hbm, msk_hbm, out_hbm,
        tx0, ty0, tz0, in0, mk0,
        tx1, ty1, tz1, in1, mk1,
        acc_v, sem0, sem1,
    ):
        cid = lax.axis_index("c")
        sid = lax.axis_index("s")
        wid = sid * _NC + cid
        cnt = (nchunk - wid + _NW - 1) // _NW

        hbms = (tx_hbm, ty_hbm, tz_hbm, inp_hbm, msk_hbm)
        sets = ((tx0, ty0, tz0, in0, mk0), (tx1, ty1, tz1, in1, mk1))
        sems = (sem0, sem1)

        iota = lax.iota(jnp.int32, _L)
        three = iota * 3
        third = jnp.float32(1.0 / 3.0)
        eps = jnp.float32(1e-6)

        def descs(k):
            c = jnp.where(k < cnt, wid + k * _NW, wid)
            rows = pl.ds(c * (_CSEG * 3), _CSEG * 3)
            return [
                pltpu.make_async_copy(h.at[rows], v, sems[k % 2])
                for h, v in zip(hbms, sets[k % 2])
            ]

        def chunk_sum(bufs):
            tx_v, ty_v, tz_v, inp_v, msk_v = bufs

            def grp_body(j, a):
                r0 = three + j * (3 * _L)
                r1 = r0 + 1
                r2 = r0 + 2
                t0x = plsc.load_gather(tx_v, [r0])
                t0y = plsc.load_gather(ty_v, [r0])
                t0z = plsc.load_gather(tz_v, [r0])
                t1x = plsc.load_gather(tx_v, [r1])
                t1y = plsc.load_gather(ty_v, [r1])
                t1z = plsc.load_gather(tz_v, [r1])
                t2x = plsc.load_gather(tx_v, [r2])
                t2y = plsc.load_gather(ty_v, [r2])
                t2z = plsc.load_gather(tz_v, [r2])
                m0 = plsc.load_gather(msk_v, [r0])
                m1 = plsc.load_gather(msk_v, [r1])
                m2 = plsc.load_gather(msk_v, [r2])
                x0 = plsc.load_gather(inp_v, [r0])
                x1 = plsc.load_gather(inp_v, [r1])
                x2 = plsc.load_gather(inp_v, [r2])

                ax = t0x - t1x
                ay = t0y - t1y
                az = t0z - t1z
                bx = t0x - t2x
                by = t0y - t2y
                bz = t0z - t2z
                cx = t1x - t2x
                cy = t1y - t2y
                cz = t1z - t2z
                d01 = _vsqrt(ax * ax + ay * ay + az * az)
                d02 = _vsqrt(bx * bx + by * by + bz * bz)
                d12 = _vsqrt(cx * cx + cy * cy + cz * cz)

                e0 = (m0 * m1) * (x0 - d01)
                e1 = (m0 * m2) * (x1 - d02)
                e2 = (m1 * m2) * (x2 - d12)
                r = e0 * e0 + e1 * e1 + e2 * e2
                return a + _vsqrt(r * third + eps)

            return lax.fori_loop(0, _GRP, grp_body, jnp.zeros((_L,), jnp.float32))

        all_descs = [descs(k) for k in range(maxc)]
        for d in all_descs[0]:
            d.start()
        acc = jnp.zeros((_L,), jnp.float32)
        for k in range(maxc):
            if k + 1 < maxc:
                for d in all_descs[k + 1]:
                    d.start()
            for d in all_descs[k]:
                d.wait()
            w = jnp.where(k < cnt, jnp.float32(1.0), jnp.float32(0.0))
            acc = acc + w * chunk_sum(sets[k % 2])
        acc_v[...] = acc
        pltpu.sync_copy(acc_v, out_hbm.at[wid])

    return sc_kernel


def kernel(inputs, target, mask, indices):
    S = indices.shape[0]
    tx = target[:, 0]
    ty = target[:, 1]
    tz = target[:, 2]
    partials = _make_sc_kernel(S)(inputs, tx, ty, tz, mask.reshape(-1))
    return jnp.sum(partials) / jnp.float32(S)
